# baseline (speedup 1.0000x reference)
import functools

import jax
import jax.numpy as jnp
from jax import lax
from jax.experimental import pallas as pl
from jax.experimental.pallas import tpu as pltpu
from jax.experimental.pallas import tpu_sc as plsc

N = 10000
E = 320000
D = 128
DH = D // 2

NC = 2
NS = 16
IDXW = 128
ROWS_PT = 160
E_PAD = NS * ROWS_PT * IDXW
CH = 2
NCHUNK = ROWS_PT // CH
ACC_ROWS = 10240
ACC_PT = ACC_ROWS // NS

_mesh = plsc.VectorSubcoreMesh(
    core_axis_name="c", subcore_axis_name="s", num_cores=NC, num_subcores=NS
)


@functools.partial(
    pl.kernel,
    out_type=jax.ShapeDtypeStruct((NC, ACC_ROWS), jnp.float32),
    mesh=_mesh,
    scratch_types=[
        pltpu.VMEM((CH, IDXW), jnp.int32),
        pltpu.VMEM((IDXW,), jnp.float32),
        pltpu.VMEM((ACC_PT,), jnp.float32),
        pltpu.VMEM_SHARED((ACC_ROWS,), jnp.float32),
    ],
    compiler_params=pltpu.CompilerParams(use_tc_tiling_on_sc=False),
)
def _sc_degree(dstp_hbm, out_hbm, dst_v, ones_v, bounce_v, acc_sh):
    c = lax.axis_index("c")
    s = lax.axis_index("s")

    for k in range(ACC_PT // 16):
        bounce_v[pl.ds(k * 16, 16)] = jnp.zeros((16,), jnp.float32)
    pltpu.sync_copy(bounce_v, acc_sh.at[pl.ds(s * ACC_PT, ACC_PT)])
    for k in range(IDXW // 16):
        ones_v[pl.ds(k * 16, 16)] = jnp.ones((16,), jnp.float32)
    plsc.subcore_barrier()

    def chunk(chi, _):
        pltpu.sync_copy(dstp_hbm.at[s, pl.ds(chi * CH, CH)], dst_v)
        for j in range(CH):
            pltpu.sync_copy(ones_v, acc_sh.at[dst_v.at[j]], add=True)
        return 0
    half = NCHUNK // NC
    lax.fori_loop(c * half, (c + 1) * half, chunk, 0)
    plsc.subcore_barrier()

    pltpu.sync_copy(acc_sh.at[pl.ds(s * ACC_PT, ACC_PT)], bounce_v)
    pltpu.sync_copy(bounce_v, out_hbm.at[c, pl.ds(s * ACC_PT, ACC_PT)])


@functools.partial(
    pl.kernel,
    out_type=jax.ShapeDtypeStruct((NC, ACC_ROWS, DH), jnp.float32),
    mesh=_mesh,
    scratch_types=[
        pltpu.VMEM((ROWS_PT, IDXW), jnp.int32),
        pltpu.VMEM((ROWS_PT, IDXW), jnp.int32),
        pltpu.VMEM((CH * IDXW, DH), jnp.float32),
        pltpu.VMEM((CH * IDXW, DH), jnp.float32),
        pltpu.VMEM_SHARED((ACC_ROWS, DH), jnp.float32),
        pltpu.SemaphoreType.DMA,
        pltpu.SemaphoreType.DMA,
        pltpu.SemaphoreType.DMA,
        pltpu.SemaphoreType.DMA,
    ],
    compiler_params=pltpu.CompilerParams(use_tc_tiling_on_sc=False),
)
def _sc_scatter(h2_hbm, srcp_hbm, dstp_hbm, out_hbm,
                srcp_v, dstp_v, rows0, rows1, acc_sh,
                sem_g0, sem_g1, sem_s0, sem_s1):
    c = lax.axis_index("c")
    s = lax.axis_index("s")
    slots = ((rows0, sem_g0, sem_s0),
             (rows1, sem_g1, sem_s1))

    pltpu.async_copy(srcp_hbm.at[c, s], srcp_v, sem_g0)
    pltpu.async_copy(dstp_hbm.at[s], dstp_v, sem_g1)

    def zfill(i, _):
        for k in range(DH // 16):
            rows0[i, pl.ds(k * 16, 16)] = jnp.zeros((16,), jnp.float32)
        return 0
    lax.fori_loop(0, IDXW, zfill, 0)
    def zcopy(i, _):
        pltpu.sync_copy(rows0.at[pl.ds(0, IDXW)],
                        acc_sh.at[pl.ds(s * ACC_PT + i * IDXW, IDXW)])
        return 0
    lax.fori_loop(0, ACC_PT // IDXW, zcopy, 0)
    pltpu.make_async_copy(srcp_hbm.at[c, s], srcp_v, sem_g0).wait()
    pltpu.make_async_copy(dstp_hbm.at[s], dstp_v, sem_g1).wait()
    plsc.subcore_barrier()

    def fire_gather(chi, slot):
        rows_v, sem_g, _ = slots[slot]
        for j in range(CH):
            pltpu.async_copy(h2_hbm.at[srcp_v.at[chi * CH + j]],
                             rows_v.at[pl.ds(j * IDXW, IDXW)], sem_g)

    def wait_gather(chi, slot):
        rows_v, sem_g, _ = slots[slot]
        for j in range(CH):
            pltpu.make_async_copy(h2_hbm.at[srcp_v.at[chi * CH + j]],
                                  rows_v.at[pl.ds(j * IDXW, IDXW)], sem_g).wait()

    def fire_scatter(chi, slot):
        rows_v, _, sem_s = slots[slot]
        for j in range(CH):
            pltpu.async_copy(rows_v.at[pl.ds(j * IDXW, IDXW)],
                             acc_sh.at[dstp_v.at[chi * CH + j]], sem_s, add=True)

    def wait_scatter(chi, slot):
        rows_v, _, sem_s = slots[slot]
        for j in range(CH):
            pltpu.make_async_copy(rows_v.at[pl.ds(j * IDXW, IDXW)],
                                  acc_sh.at[dstp_v.at[chi * CH + j]], sem_s).wait()

    fire_gather(0, 0)
    fire_gather(1, 1)

    def body(i, _):
        a = 2 * i
        wait_gather(a, 0)
        fire_scatter(a, 0)
        wait_gather(a + 1, 1)
        fire_scatter(a + 1, 1)
        wait_scatter(a, 0)
        fire_gather(a + 2, 0)
        wait_scatter(a + 1, 1)
        fire_gather(a + 3, 1)
        return 0
    lax.fori_loop(0, NCHUNK // 2 - 1, body, 0)

    last = NCHUNK - 2
    wait_gather(last, 0)
    fire_scatter(last, 0)
    wait_gather(last + 1, 1)
    fire_scatter(last + 1, 1)
    wait_scatter(last, 0)
    wait_scatter(last + 1, 1)
    plsc.subcore_barrier()

    for i in range(ACC_PT // IDXW):
        buf, _, sem = slots[i % 2]
        if i >= 2:
            pltpu.make_async_copy(
                buf.at[pl.ds(0, IDXW)],
                out_hbm.at[c, pl.ds(s * ACC_PT + (i - 2) * IDXW, IDXW)],
                sem).wait()
        pltpu.sync_copy(acc_sh.at[pl.ds(s * ACC_PT + i * IDXW, IDXW)],
                        buf.at[pl.ds(0, IDXW)])
        pltpu.async_copy(buf.at[pl.ds(0, IDXW)],
                         out_hbm.at[c, pl.ds(s * ACC_PT + i * IDXW, IDXW)], sem)
    for i in range(ACC_PT // IDXW - 2, ACC_PT // IDXW):
        buf, _, sem = slots[i % 2]
        pltpu.make_async_copy(
            buf.at[pl.ds(0, IDXW)],
            out_hbm.at[c, pl.ds(s * ACC_PT + i * IDXW, IDXW)], sem).wait()


BLK = 2048


def _dinv_of(degp_ref):
    deg = degp_ref[0, :] + degp_ref[1, :] + 1.0
    return lax.rsqrt(deg)


def _s_full(s_ref):
    return jnp.concatenate([s_ref[0], s_ref[1]], axis=-1)


def _tc_pre_body(x_ref, w_ref, degp_ref, out_ref):
    dinv = _dinv_of(degp_ref)
    h = jnp.dot(x_ref[...], w_ref[...], preferred_element_type=jnp.float32)
    out_ref[...] = h * dinv[:, None]


def _tc_mid_body(s_ref, h_ref, degp_ref, b_ref, w_ref, out_ref):
    dinv = _dinv_of(degp_ref)
    agg = (_s_full(s_ref) + h_ref[...]) * dinv[:, None] + b_ref[...]
    z = jnp.maximum(agg, 0.0)
    out_ref[...] = jnp.dot(z, w_ref[...], preferred_element_type=jnp.float32) * dinv[:, None]


def _tc_final_body(s_ref, h_ref, degp_ref, b_ref, wp_ref, bp_ref, out_ref):
    dinv = _dinv_of(degp_ref)
    agg = (_s_full(s_ref) + h_ref[...]) * dinv[:, None] + b_ref[...]
    z = jnp.maximum(agg, 0.0)
    out_ref[...] = jnp.dot(z, wp_ref[...], preferred_element_type=jnp.float32) + bp_ref[...]


_spec_rows = pl.BlockSpec((BLK, D), lambda i: (i, 0))
_spec_w = pl.BlockSpec((D, D), lambda i: (0, 0))
_spec_b = pl.BlockSpec((1, D), lambda i: (0, 0))
_spec_degp = pl.BlockSpec((NC, BLK), lambda i: (0, i))
_spec_s = pl.BlockSpec((NC, BLK, DH), lambda i: (0, i, 0))
_out_rows = jax.ShapeDtypeStruct((ACC_ROWS, D), jnp.float32)

_tc_pre = pl.pallas_call(
    _tc_pre_body, grid=(ACC_ROWS // BLK,),
    in_specs=[_spec_rows, _spec_w, _spec_degp],
    out_specs=_spec_rows, out_shape=_out_rows,
)
_tc_mid = pl.pallas_call(
    _tc_mid_body, grid=(ACC_ROWS // BLK,),
    in_specs=[_spec_s, _spec_rows, _spec_degp, _spec_b, _spec_w],
    out_specs=_spec_rows, out_shape=_out_rows,
)
_tc_final = pl.pallas_call(
    _tc_final_body, grid=(ACC_ROWS // BLK,),
    in_specs=[_spec_s, _spec_rows, _spec_degp, _spec_b, _spec_w, _spec_b],
    out_specs=_spec_rows, out_shape=_out_rows,
)


def kernel(x, edge_index, W0, b0, W1, b1, W2, b2, Wp, bp):
    src = edge_index[0].astype(jnp.int32)
    dst = edge_index[1].astype(jnp.int32)
    pad = E_PAD - E
    srcp = jnp.concatenate([src, jnp.zeros((pad,), jnp.int32)]).reshape(NS, ROWS_PT, IDXW)
    srcp2 = jnp.stack([2 * srcp, 2 * srcp + 1])
    dstp = jnp.concatenate([dst, jnp.full((pad,), N, jnp.int32)]).reshape(NS, ROWS_PT, IDXW)

    degp = _sc_degree(dstp)

    xp = jnp.concatenate([x, jnp.zeros((ACC_ROWS - N, D), x.dtype)])
    b0r = b0.reshape(1, D)
    b1r = b1.reshape(1, D)
    b2r = b2.reshape(1, D)
    bpr = bp.reshape(1, D)

    h0 = _tc_pre(xp, W0, degp)
    s0 = _sc_scatter(h0.reshape(2 * ACC_ROWS, DH), srcp2, dstp)
    h1 = _tc_mid(s0, h0, degp, b0r, W1)
    s1 = _sc_scatter(h1.reshape(2 * ACC_ROWS, DH), srcp2, dstp)
    h2 = _tc_mid(s1, h1, degp, b1r, W2)
    s2 = _sc_scatter(h2.reshape(2 * ACC_ROWS, DH), srcp2, dstp)
    out = _tc_final(s2, h2, degp, b2r, Wp, bpr)
    return out[:N]


# speedup vs baseline: 1.0335x; 1.0335x over previous
import functools

import jax
import jax.numpy as jnp
from jax import lax
from jax.experimental import pallas as pl
from jax.experimental.pallas import tpu as pltpu
from jax.experimental.pallas import tpu_sc as plsc

N = 10000
E = 320000
D = 128
DH = D // 2

NC = 2
NS = 16
IDXW = 128
ROWS_PT = 160
E_PAD = NS * ROWS_PT * IDXW
CH = 5
NCHUNK = ROWS_PT // CH
ACC_ROWS = 10240
ACC_PT = ACC_ROWS // NS

_mesh = plsc.VectorSubcoreMesh(
    core_axis_name="c", subcore_axis_name="s", num_cores=NC, num_subcores=NS
)


@functools.partial(
    pl.kernel,
    out_type=jax.ShapeDtypeStruct((NC, ACC_ROWS), jnp.float32),
    mesh=_mesh,
    scratch_types=[
        pltpu.VMEM((CH, IDXW), jnp.int32),
        pltpu.VMEM((IDXW,), jnp.float32),
        pltpu.VMEM((ACC_PT,), jnp.float32),
        pltpu.VMEM_SHARED((ACC_ROWS,), jnp.float32),
    ],
    compiler_params=pltpu.CompilerParams(use_tc_tiling_on_sc=False),
)
def _sc_degree(dstp_hbm, out_hbm, dst_v, ones_v, bounce_v, acc_sh):
    c = lax.axis_index("c")
    s = lax.axis_index("s")

    for k in range(ACC_PT // 16):
        bounce_v[pl.ds(k * 16, 16)] = jnp.zeros((16,), jnp.float32)
    pltpu.sync_copy(bounce_v, acc_sh.at[pl.ds(s * ACC_PT, ACC_PT)])
    for k in range(IDXW // 16):
        ones_v[pl.ds(k * 16, 16)] = jnp.ones((16,), jnp.float32)
    plsc.subcore_barrier()

    def chunk(chi, _):
        pltpu.sync_copy(dstp_hbm.at[s, pl.ds(chi * CH, CH)], dst_v)
        for j in range(CH):
            pltpu.sync_copy(ones_v, acc_sh.at[dst_v.at[j]], add=True)
        return 0
    half = NCHUNK // NC
    lax.fori_loop(c * half, (c + 1) * half, chunk, 0)
    plsc.subcore_barrier()

    pltpu.sync_copy(acc_sh.at[pl.ds(s * ACC_PT, ACC_PT)], bounce_v)
    pltpu.sync_copy(bounce_v, out_hbm.at[c, pl.ds(s * ACC_PT, ACC_PT)])


@functools.partial(
    pl.kernel,
    out_type=jax.ShapeDtypeStruct((NC, ACC_ROWS, DH), jnp.float32),
    mesh=_mesh,
    scratch_types=[
        pltpu.VMEM((2 * CH, IDXW), jnp.int32),
        pltpu.VMEM((2 * CH, IDXW), jnp.int32),
        pltpu.VMEM((CH * IDXW, DH), jnp.float32),
        pltpu.VMEM((CH * IDXW, DH), jnp.float32),
        pltpu.VMEM_SHARED((ACC_ROWS, DH), jnp.float32),
        pltpu.SemaphoreType.DMA,
        pltpu.SemaphoreType.DMA,
        pltpu.SemaphoreType.DMA,
        pltpu.SemaphoreType.DMA,
    ],
    compiler_params=pltpu.CompilerParams(use_tc_tiling_on_sc=False),
)
def _sc_scatter(h2_hbm, srcp_hbm, dstp_hbm, out_hbm,
                srcp_v, dstp_v, rows0, rows1, acc_sh,
                sem_g0, sem_g1, sem_s0, sem_s1):
    c = lax.axis_index("c")
    s = lax.axis_index("s")
    slots = ((rows0, sem_g0, sem_s0),
             (rows1, sem_g1, sem_s1))

    def zfill(i, _):
        for k in range(DH // 16):
            rows0[i, pl.ds(k * 16, 16)] = jnp.zeros((16,), jnp.float32)
        return 0
    lax.fori_loop(0, IDXW, zfill, 0)
    def zcopy(i, _):
        pltpu.sync_copy(rows0.at[pl.ds(0, IDXW)],
                        acc_sh.at[pl.ds(s * ACC_PT + i * IDXW, IDXW)])
        return 0
    lax.fori_loop(0, ACC_PT // IDXW, zcopy, 0)
    plsc.subcore_barrier()

    def load_and_gather(chi, slot):
        rows_v, sem_g, _ = slots[slot]
        pltpu.sync_copy(srcp_hbm.at[c, s, pl.ds(chi * CH, CH)],
                        srcp_v.at[pl.ds(slot * CH, CH)])
        pltpu.sync_copy(dstp_hbm.at[s, pl.ds(chi * CH, CH)],
                        dstp_v.at[pl.ds(slot * CH, CH)])
        for j in range(CH):
            pltpu.async_copy(h2_hbm.at[srcp_v.at[slot * CH + j]],
                             rows_v.at[pl.ds(j * IDXW, IDXW)], sem_g)

    def wait_gather(slot):
        rows_v, sem_g, _ = slots[slot]
        for j in range(CH):
            pltpu.make_async_copy(h2_hbm.at[srcp_v.at[slot * CH + j]],
                                  rows_v.at[pl.ds(j * IDXW, IDXW)], sem_g).wait()

    def fire_scatter(slot):
        rows_v, _, sem_s = slots[slot]
        for j in range(CH):
            pltpu.async_copy(rows_v.at[pl.ds(j * IDXW, IDXW)],
                             acc_sh.at[dstp_v.at[slot * CH + j]], sem_s, add=True)

    def wait_scatter(slot):
        rows_v, _, sem_s = slots[slot]
        for j in range(CH):
            pltpu.make_async_copy(rows_v.at[pl.ds(j * IDXW, IDXW)],
                                  acc_sh.at[dstp_v.at[slot * CH + j]], sem_s).wait()

    load_and_gather(0, 0)
    load_and_gather(1, 1)

    def body(i, _):
        a = 2 * i + 2
        wait_gather(0)
        fire_scatter(0)
        wait_gather(1)
        fire_scatter(1)
        wait_scatter(0)
        load_and_gather(a, 0)
        wait_scatter(1)
        load_and_gather(a + 1, 1)
        return 0
    lax.fori_loop(0, NCHUNK // 2 - 1, body, 0)

    wait_gather(0)
    fire_scatter(0)
    wait_gather(1)
    fire_scatter(1)
    wait_scatter(0)
    wait_scatter(1)
    plsc.subcore_barrier()

    for i in range(ACC_PT // IDXW):
        buf, _, sem = slots[i % 2]
        if i >= 2:
            pltpu.make_async_copy(
                buf.at[pl.ds(0, IDXW)],
                out_hbm.at[c, pl.ds(s * ACC_PT + (i - 2) * IDXW, IDXW)],
                sem).wait()
        pltpu.sync_copy(acc_sh.at[pl.ds(s * ACC_PT + i * IDXW, IDXW)],
                        buf.at[pl.ds(0, IDXW)])
        pltpu.async_copy(buf.at[pl.ds(0, IDXW)],
                         out_hbm.at[c, pl.ds(s * ACC_PT + i * IDXW, IDXW)], sem)
    for i in range(ACC_PT // IDXW - 2, ACC_PT // IDXW):
        buf, _, sem = slots[i % 2]
        pltpu.make_async_copy(
            buf.at[pl.ds(0, IDXW)],
            out_hbm.at[c, pl.ds(s * ACC_PT + i * IDXW, IDXW)], sem).wait()


BLK = 2048


def _dinv_of(degp_ref):
    deg = degp_ref[0, :] + degp_ref[1, :] + 1.0
    return lax.rsqrt(deg)


def _s_full(s_ref):
    return jnp.concatenate([s_ref[0], s_ref[1]], axis=-1)


def _tc_pre_body(x_ref, w_ref, degp_ref, out_ref):
    dinv = _dinv_of(degp_ref)
    h = jnp.dot(x_ref[...], w_ref[...], preferred_element_type=jnp.float32)
    out_ref[...] = h * dinv[:, None]


def _tc_mid_body(s_ref, h_ref, degp_ref, b_ref, w_ref, out_ref):
    dinv = _dinv_of(degp_ref)
    agg = (_s_full(s_ref) + h_ref[...]) * dinv[:, None] + b_ref[...]
    z = jnp.maximum(agg, 0.0)
    out_ref[...] = jnp.dot(z, w_ref[...], preferred_element_type=jnp.float32) * dinv[:, None]


def _tc_final_body(s_ref, h_ref, degp_ref, b_ref, wp_ref, bp_ref, out_ref):
    dinv = _dinv_of(degp_ref)
    agg = (_s_full(s_ref) + h_ref[...]) * dinv[:, None] + b_ref[...]
    z = jnp.maximum(agg, 0.0)
    out_ref[...] = jnp.dot(z, wp_ref[...], preferred_element_type=jnp.float32) + bp_ref[...]


_spec_rows = pl.BlockSpec((BLK, D), lambda i: (i, 0))
_spec_w = pl.BlockSpec((D, D), lambda i: (0, 0))
_spec_b = pl.BlockSpec((1, D), lambda i: (0, 0))
_spec_degp = pl.BlockSpec((NC, BLK), lambda i: (0, i))
_spec_s = pl.BlockSpec((NC, BLK, DH), lambda i: (0, i, 0))
_out_rows = jax.ShapeDtypeStruct((ACC_ROWS, D), jnp.float32)

_tc_pre = pl.pallas_call(
    _tc_pre_body, grid=(ACC_ROWS // BLK,),
    in_specs=[_spec_rows, _spec_w, _spec_degp],
    out_specs=_spec_rows, out_shape=_out_rows,
)
_tc_mid = pl.pallas_call(
    _tc_mid_body, grid=(ACC_ROWS // BLK,),
    in_specs=[_spec_s, _spec_rows, _spec_degp, _spec_b, _spec_w],
    out_specs=_spec_rows, out_shape=_out_rows,
)
_tc_final = pl.pallas_call(
    _tc_final_body, grid=(ACC_ROWS // BLK,),
    in_specs=[_spec_s, _spec_rows, _spec_degp, _spec_b, _spec_w, _spec_b],
    out_specs=_spec_rows, out_shape=_out_rows,
)


def kernel(x, edge_index, W0, b0, W1, b1, W2, b2, Wp, bp):
    src = edge_index[0].astype(jnp.int32)
    dst = edge_index[1].astype(jnp.int32)
    pad = E_PAD - E
    srcp = jnp.concatenate([src, jnp.zeros((pad,), jnp.int32)]).reshape(NS, ROWS_PT, IDXW)
    srcp2 = jnp.stack([2 * srcp, 2 * srcp + 1])
    dstp = jnp.concatenate([dst, jnp.full((pad,), N, jnp.int32)]).reshape(NS, ROWS_PT, IDXW)

    degp = _sc_degree(dstp)

    xp = jnp.concatenate([x, jnp.zeros((ACC_ROWS - N, D), x.dtype)])
    b0r = b0.reshape(1, D)
    b1r = b1.reshape(1, D)
    b2r = b2.reshape(1, D)
    bpr = bp.reshape(1, D)

    h0 = _tc_pre(xp, W0, degp)
    s0 = _sc_scatter(h0.reshape(2 * ACC_ROWS, DH), srcp2, dstp)
    h1 = _tc_mid(s0, h0, degp, b0r, W1)
    s1 = _sc_scatter(h1.reshape(2 * ACC_ROWS, DH), srcp2, dstp)
    h2 = _tc_mid(s1, h1, degp, b1r, W2)
    s2 = _sc_scatter(h2.reshape(2 * ACC_ROWS, DH), srcp2, dstp)
    out = _tc_final(s2, h2, degp, b2r, Wp, bpr)
    return out[:N]


# speedup vs baseline: 1.0368x; 1.0032x over previous
import functools

import jax
import jax.numpy as jnp
from jax import lax
from jax.experimental import pallas as pl
from jax.experimental.pallas import tpu as pltpu
from jax.experimental.pallas import tpu_sc as plsc

N = 10000
E = 320000
D = 128
DH = D // 2

NC = 2
NS = 16
IDXW = 128
ROWS_PT = 160
E_PAD = NS * ROWS_PT * IDXW
CH = 5
NCHUNK = ROWS_PT // CH
ACC_ROWS = 10240
ACC_PT = ACC_ROWS // NS

_mesh = plsc.VectorSubcoreMesh(
    core_axis_name="c", subcore_axis_name="s", num_cores=NC, num_subcores=NS
)


@functools.partial(
    pl.kernel,
    out_type=jax.ShapeDtypeStruct((NC, ACC_ROWS), jnp.float32),
    mesh=_mesh,
    scratch_types=[
        pltpu.VMEM((CH, IDXW), jnp.int32),
        pltpu.VMEM((IDXW,), jnp.float32),
        pltpu.VMEM((ACC_PT,), jnp.float32),
        pltpu.VMEM_SHARED((ACC_ROWS,), jnp.float32),
    ],
    compiler_params=pltpu.CompilerParams(use_tc_tiling_on_sc=False),
)
def _sc_degree(dstp_hbm, out_hbm, dst_v, ones_v, bounce_v, acc_sh):
    c = lax.axis_index("c")
    s = lax.axis_index("s")

    for k in range(ACC_PT // 16):
        bounce_v[pl.ds(k * 16, 16)] = jnp.zeros((16,), jnp.float32)
    pltpu.sync_copy(bounce_v, acc_sh.at[pl.ds(s * ACC_PT, ACC_PT)])
    for k in range(IDXW // 16):
        ones_v[pl.ds(k * 16, 16)] = jnp.ones((16,), jnp.float32)
    plsc.subcore_barrier()

    def chunk(chi, _):
        pltpu.sync_copy(dstp_hbm.at[s, pl.ds(chi * CH, CH)], dst_v)
        for j in range(CH):
            pltpu.sync_copy(ones_v, acc_sh.at[dst_v.at[j]], add=True)
        return 0
    half = NCHUNK // NC
    lax.fori_loop(c * half, (c + 1) * half, chunk, 0)
    plsc.subcore_barrier()

    pltpu.sync_copy(acc_sh.at[pl.ds(s * ACC_PT, ACC_PT)], bounce_v)
    pltpu.sync_copy(bounce_v, out_hbm.at[c, pl.ds(s * ACC_PT, ACC_PT)])


@functools.partial(
    pl.kernel,
    out_type=jax.ShapeDtypeStruct((NC, ACC_ROWS, DH), jnp.float32),
    mesh=_mesh,
    scratch_types=[
        pltpu.VMEM((2 * CH, IDXW), jnp.int32),
        pltpu.VMEM((2 * CH, IDXW), jnp.int32),
        pltpu.VMEM((CH * IDXW, DH), jnp.float32),
        pltpu.VMEM((CH * IDXW, DH), jnp.float32),
        pltpu.VMEM_SHARED((ACC_ROWS, DH), jnp.float32),
        pltpu.SemaphoreType.DMA,
        pltpu.SemaphoreType.DMA,
        pltpu.SemaphoreType.DMA,
        pltpu.SemaphoreType.DMA,
    ],
    compiler_params=pltpu.CompilerParams(use_tc_tiling_on_sc=False),
)
def _sc_scatter(h2_hbm, srcp_hbm, dstp_hbm, out_hbm,
                srcp_v, dstp_v, rows0, rows1, acc_sh,
                sem_g0, sem_g1, sem_s0, sem_s1):
    c = lax.axis_index("c")
    s = lax.axis_index("s")
    slots = ((rows0, sem_g0, sem_s0),
             (rows1, sem_g1, sem_s1))

    def zfill(i, _):
        for k in range(DH // 16):
            rows0[i, pl.ds(k * 16, 16)] = jnp.zeros((16,), jnp.float32)
        return 0
    lax.fori_loop(0, IDXW, zfill, 0)
    def zcopy(i, _):
        pltpu.sync_copy(rows0.at[pl.ds(0, IDXW)],
                        acc_sh.at[pl.ds(s * ACC_PT + i * IDXW, IDXW)])
        return 0
    lax.fori_loop(0, ACC_PT // IDXW, zcopy, 0)
    plsc.subcore_barrier()

    def load_and_gather(chi, slot):
        rows_v, sem_g, _ = slots[slot]
        pltpu.sync_copy(srcp_hbm.at[c, s, pl.ds(chi * CH, CH)],
                        srcp_v.at[pl.ds(slot * CH, CH)])
        pltpu.sync_copy(dstp_hbm.at[s, pl.ds(chi * CH, CH)],
                        dstp_v.at[pl.ds(slot * CH, CH)])
        for j in range(CH):
            pltpu.async_copy(h2_hbm.at[srcp_v.at[slot * CH + j]],
                             rows_v.at[pl.ds(j * IDXW, IDXW)], sem_g)

    def wait_gather(slot):
        rows_v, sem_g, _ = slots[slot]
        for j in range(CH):
            pltpu.make_async_copy(h2_hbm.at[srcp_v.at[slot * CH + j]],
                                  rows_v.at[pl.ds(j * IDXW, IDXW)], sem_g).wait()

    def fire_scatter(slot):
        rows_v, _, sem_s = slots[slot]
        for j in range(CH):
            pltpu.async_copy(rows_v.at[pl.ds(j * IDXW, IDXW)],
                             acc_sh.at[dstp_v.at[slot * CH + j]], sem_s, add=True)

    def wait_scatter(slot):
        rows_v, _, sem_s = slots[slot]
        for j in range(CH):
            pltpu.make_async_copy(rows_v.at[pl.ds(j * IDXW, IDXW)],
                                  acc_sh.at[dstp_v.at[slot * CH + j]], sem_s).wait()

    load_and_gather(0, 0)
    load_and_gather(1, 1)

    def body(i, _):
        a = 2 * i + 2
        wait_gather(0)
        fire_scatter(0)
        wait_gather(1)
        fire_scatter(1)
        wait_scatter(0)
        load_and_gather(a, 0)
        wait_scatter(1)
        load_and_gather(a + 1, 1)
        return 0
    lax.fori_loop(0, NCHUNK // 2 - 1, body, 0)

    wait_gather(0)
    fire_scatter(0)
    wait_gather(1)
    fire_scatter(1)
    wait_scatter(0)
    wait_scatter(1)
    plsc.subcore_barrier()

    for i in range(ACC_PT // IDXW):
        buf, _, sem = slots[i % 2]
        if i >= 2:
            pltpu.make_async_copy(
                buf.at[pl.ds(0, IDXW)],
                out_hbm.at[c, pl.ds(s * ACC_PT + (i - 2) * IDXW, IDXW)],
                sem).wait()
        pltpu.sync_copy(acc_sh.at[pl.ds(s * ACC_PT + i * IDXW, IDXW)],
                        buf.at[pl.ds(0, IDXW)])
        pltpu.async_copy(buf.at[pl.ds(0, IDXW)],
                         out_hbm.at[c, pl.ds(s * ACC_PT + i * IDXW, IDXW)], sem)
    for i in range(ACC_PT // IDXW - 2, ACC_PT // IDXW):
        buf, _, sem = slots[i % 2]
        pltpu.make_async_copy(
            buf.at[pl.ds(0, IDXW)],
            out_hbm.at[c, pl.ds(s * ACC_PT + i * IDXW, IDXW)], sem).wait()


BLK = 2560


def _dinv_of(degp_ref):
    deg = degp_ref[0, :] + degp_ref[1, :] + 1.0
    return lax.rsqrt(deg)


def _s_full(s_ref):
    return jnp.concatenate([s_ref[0], s_ref[1]], axis=-1)


def _tc_pre_body(x_ref, w_ref, degp_ref, out_ref):
    dinv = _dinv_of(degp_ref)
    h = jnp.dot(x_ref[...], w_ref[...], preferred_element_type=jnp.float32)
    out_ref[...] = h * dinv[:, None]


def _tc_mid_body(s_ref, h_ref, degp_ref, b_ref, w_ref, out_ref):
    dinv = _dinv_of(degp_ref)
    agg = (_s_full(s_ref) + h_ref[...]) * dinv[:, None] + b_ref[...]
    z = jnp.maximum(agg, 0.0)
    out_ref[...] = jnp.dot(z, w_ref[...], preferred_element_type=jnp.float32) * dinv[:, None]


def _tc_final_body(s_ref, h_ref, degp_ref, b_ref, wp_ref, bp_ref, out_ref):
    dinv = _dinv_of(degp_ref)
    agg = (_s_full(s_ref) + h_ref[...]) * dinv[:, None] + b_ref[...]
    z = jnp.maximum(agg, 0.0)
    out_ref[...] = jnp.dot(z, wp_ref[...], preferred_element_type=jnp.float32) + bp_ref[...]


_spec_rows = pl.BlockSpec((BLK, D), lambda i: (i, 0))
_spec_w = pl.BlockSpec((D, D), lambda i: (0, 0))
_spec_b = pl.BlockSpec((1, D), lambda i: (0, 0))
_spec_degp = pl.BlockSpec((NC, BLK), lambda i: (0, i))
_spec_s = pl.BlockSpec((NC, BLK, DH), lambda i: (0, i, 0))
_out_rows = jax.ShapeDtypeStruct((ACC_ROWS, D), jnp.float32)

_tc_pre = pl.pallas_call(
    _tc_pre_body, grid=(ACC_ROWS // BLK,),
    in_specs=[_spec_rows, _spec_w, _spec_degp],
    out_specs=_spec_rows, out_shape=_out_rows,
)
_tc_mid = pl.pallas_call(
    _tc_mid_body, grid=(ACC_ROWS // BLK,),
    in_specs=[_spec_s, _spec_rows, _spec_degp, _spec_b, _spec_w],
    out_specs=_spec_rows, out_shape=_out_rows,
)
_tc_final = pl.pallas_call(
    _tc_final_body, grid=(ACC_ROWS // BLK,),
    in_specs=[_spec_s, _spec_rows, _spec_degp, _spec_b, _spec_w, _spec_b],
    out_specs=_spec_rows, out_shape=_out_rows,
)


def kernel(x, edge_index, W0, b0, W1, b1, W2, b2, Wp, bp):
    src = edge_index[0].astype(jnp.int32)
    dst = edge_index[1].astype(jnp.int32)
    pad = E_PAD - E
    srcp = jnp.concatenate([src, jnp.zeros((pad,), jnp.int32)]).reshape(NS, ROWS_PT, IDXW)
    srcp2 = jnp.stack([2 * srcp, 2 * srcp + 1])
    dstp = jnp.concatenate([dst, jnp.full((pad,), N, jnp.int32)]).reshape(NS, ROWS_PT, IDXW)

    degp = _sc_degree(dstp)

    xp = jnp.concatenate([x, jnp.zeros((ACC_ROWS - N, D), x.dtype)])
    b0r = b0.reshape(1, D)
    b1r = b1.reshape(1, D)
    b2r = b2.reshape(1, D)
    bpr = bp.reshape(1, D)

    h0 = _tc_pre(xp, W0, degp)
    s0 = _sc_scatter(h0.reshape(2 * ACC_ROWS, DH), srcp2, dstp)
    h1 = _tc_mid(s0, h0, degp, b0r, W1)
    s1 = _sc_scatter(h1.reshape(2 * ACC_ROWS, DH), srcp2, dstp)
    h2 = _tc_mid(s1, h1, degp, b1r, W2)
    s2 = _sc_scatter(h2.reshape(2 * ACC_ROWS, DH), srcp2, dstp)
    out = _tc_final(s2, h2, degp, b2r, Wp, bpr)
    return out[:N]
